# Initial kernel scaffold; baseline (speedup 1.0000x reference)
import jax
import jax.numpy as jnp
from jax.experimental import pallas as pl


def kernel(hidden_states, gate, gate_up_proj, down_proj):
    raise NotImplementedError("write your pallas kernel here")


# speedup vs baseline: 1.4180x; 1.4180x over previous
import jax
import jax.numpy as jnp
from jax.experimental import pallas as pl
from jax.experimental.pallas import tpu as pltpu


def _moe_body(hs_ref, gate_ref, gu_ref, dn_ref, out_ref, w_ref):
    e = pl.program_id(0)
    T, E = w_ref.shape
    F = dn_ref.shape[1]

    @pl.when(e == 0)
    def _router():
        x = hs_ref[...]
        logits = jnp.dot(x, gate_ref[...], preferred_element_type=jnp.float32)
        p = jax.nn.softmax(logits, axis=-1)
        idx = jax.lax.broadcasted_iota(jnp.int32, p.shape, 1)
        m1 = jnp.max(p, axis=1, keepdims=True)
        i1 = jnp.min(jnp.where(p == m1, idx, E), axis=1, keepdims=True)
        sel1 = idx == i1
        p2 = jnp.where(sel1, -jnp.inf, p)
        m2 = jnp.max(p2, axis=1, keepdims=True)
        i2 = jnp.min(jnp.where(p2 == m2, idx, E), axis=1, keepdims=True)
        sel2 = idx == i2
        wsum = m1 + m2
        w = jnp.where(sel1, m1, jnp.where(sel2, m2, 0.0)) / wsum
        w_ref[...] = w
        out_ref[...] = jnp.zeros_like(out_ref)

    x = hs_ref[...]
    gu = jnp.dot(x, gu_ref[0], preferred_element_type=jnp.float32)
    g = gu[:, :F]
    u = gu[:, F:]
    act = u * (g * jax.nn.sigmoid(g))
    d = jnp.dot(act, dn_ref[0], preferred_element_type=jnp.float32)
    lane = jax.lax.broadcasted_iota(jnp.int32, (T, E), 1)
    w_col = jnp.sum(jnp.where(lane == e, w_ref[...], 0.0), axis=1, keepdims=True)
    out_ref[...] += w_col * d


def kernel(hidden_states, gate, gate_up_proj, down_proj):
    B, S, D = hidden_states.shape
    E, _, F2 = gate_up_proj.shape
    F = F2 // 2
    hs = hidden_states.reshape(-1, D)
    T = hs.shape[0]

    out = pl.pallas_call(
        _moe_body,
        grid=(E,),
        in_specs=[
            pl.BlockSpec((T, D), lambda e: (0, 0)),
            pl.BlockSpec((D, E), lambda e: (0, 0)),
            pl.BlockSpec((1, D, F2), lambda e: (e, 0, 0)),
            pl.BlockSpec((1, F, D), lambda e: (e, 0, 0)),
        ],
        out_specs=pl.BlockSpec((T, D), lambda e: (0, 0)),
        out_shape=jax.ShapeDtypeStruct((T, D), jnp.float32),
        scratch_shapes=[pltpu.VMEM((T, E), jnp.float32)],
        compiler_params=pltpu.CompilerParams(
            dimension_semantics=("arbitrary",),
        ),
    )(hs, gate, gate_up_proj, down_proj)
    return out.reshape(B, S, D)
